# baseline (speedup 1.0000x reference)
import functools

import jax
import jax.numpy as jnp
from jax import lax
from jax.experimental import pallas as pl
from jax.experimental.pallas import tpu as pltpu
from jax.experimental.pallas import tpu_sc as plsc

B = 4096
IN = 2048
H1 = 512
H2 = 256
P = 16
K = 128
D = 64

BLK = 256
GRID = B // BLK

TABW = 256
NC = 2
NS = 16
NW = NC * NS
BPW = B // NW


def _sc_gather_body(tab_hbm, idx_hbm, out_hbm, idx_v, rows_v, sem):
    wid = lax.axis_index("s") * NC + lax.axis_index("c")
    base = wid * BPW
    pltpu.sync_copy(idx_hbm.at[pl.ds(base, BPW)], idx_v)
    pltpu.async_copy(tab_hbm.at[idx_v], rows_v, sem).wait()
    pltpu.sync_copy(rows_v, out_hbm.at[pl.ds(base, BPW)])


def _route_gather(tab, drug_indices):
    mesh = plsc.VectorSubcoreMesh(core_axis_name="c", subcore_axis_name="s")
    return pl.kernel(
        _sc_gather_body,
        mesh=mesh,
        out_type=jax.ShapeDtypeStruct((B, TABW), jnp.float32),
        scratch_types=[
            pltpu.VMEM((BPW,), jnp.int32),
            pltpu.VMEM((BPW, TABW), jnp.float32),
            pltpu.SemaphoreType.DMA,
        ],
    )(tab, drug_indices)


def _fused_body(x_ref, w1_ref, b1_ref, w2_ref, b2_ref, wp_ref, bp_ref,
                g_ref, out_ref, w1b_ref, w2b_ref, wpf_ref):
    i = pl.program_id(0)

    @pl.when(i == 0)
    def _build_weights():
        w1b_ref[...] = w1_ref[...].astype(jnp.bfloat16)
        w2b_ref[...] = w2_ref[...].astype(jnp.bfloat16)
        for p in range(P):
            wpf_ref[:, p * K:(p + 1) * K] = (
                wp_ref[p].astype(jnp.bfloat16))

    x = x_ref[...].astype(jnp.bfloat16)
    h = jnp.maximum(jnp.dot(x, w1b_ref[...],
                            preferred_element_type=jnp.float32)
                    + b1_ref[0, :], 0.0)
    h = jnp.maximum(jnp.dot(h.astype(jnp.bfloat16), w2b_ref[...],
                            preferred_element_type=jnp.float32)
                    + b2_ref[0, :], 0.0)
    a = jnp.maximum(jnp.dot(h.astype(jnp.bfloat16), wpf_ref[...],
                            preferred_element_type=jnp.float32)
                    + bp_ref[...].reshape(1, P * K), 0.0)

    g = g_ref[...]
    wdg = g[:, :K]
    bdg = g[:, K]
    pwf = g[:, K + 1]

    sel = a[:, :K]
    for p in range(1, P):
        sel = jnp.where((pwf == float(p))[:, None],
                        a[:, p * K:(p + 1) * K], sel)
    out_ref[0, 0, :] = jnp.sum(sel * wdg, axis=1) + bdg


def kernel(x, drug_indices, drug_to_pw, W1, b1, W2, b2, Wp, bp, Wd, bd):
    tab = jnp.concatenate(
        [Wd, bd[:, None], drug_to_pw.astype(jnp.float32)[:, None],
         jnp.zeros((D, TABW - K - 2), jnp.float32)], axis=1)
    g = _route_gather(tab, drug_indices)

    out = pl.pallas_call(
        _fused_body,
        grid=(GRID,),
        in_specs=[
            pl.BlockSpec((BLK, IN), lambda i: (i, 0)),
            pl.BlockSpec((IN, H1), lambda i: (0, 0)),
            pl.BlockSpec((1, H1), lambda i: (0, 0)),
            pl.BlockSpec((H1, H2), lambda i: (0, 0)),
            pl.BlockSpec((1, H2), lambda i: (0, 0)),
            pl.BlockSpec((P, H2, K), lambda i: (0, 0, 0)),
            pl.BlockSpec((P, K), lambda i: (0, 0)),
            pl.BlockSpec((BLK, TABW), lambda i: (i, 0)),
        ],
        out_specs=pl.BlockSpec((1, 1, BLK), lambda i: (i, 0, 0)),
        out_shape=jax.ShapeDtypeStruct((GRID, 1, BLK), jnp.float32),
        scratch_shapes=[
            pltpu.VMEM((IN, H1), jnp.bfloat16),
            pltpu.VMEM((H1, H2), jnp.bfloat16),
            pltpu.VMEM((H2, P * K), jnp.bfloat16),
        ],
    )(x, W1, b1.reshape(1, H1), W2, b2.reshape(1, H2), Wp, bp, g)
    return out.reshape(B)


# speedup vs baseline: 1.0622x; 1.0622x over previous
import functools

import jax
import jax.numpy as jnp
from jax import lax
from jax.experimental import pallas as pl
from jax.experimental.pallas import tpu as pltpu
from jax.experimental.pallas import tpu_sc as plsc

B = 4096
IN = 2048
H1 = 512
H2 = 256
P = 16
K = 128
D = 64

BLK = 512
GRID = B // BLK

TABW = 256
NC = 2
NS = 16
NW = NC * NS
BPW = B // NW


def _sc_gather_body(tab_hbm, idx_hbm, out_hbm, idx_v, rows_v, sem):
    wid = lax.axis_index("s") * NC + lax.axis_index("c")
    base = wid * BPW
    pltpu.sync_copy(idx_hbm.at[pl.ds(base, BPW)], idx_v)
    pltpu.async_copy(tab_hbm.at[idx_v], rows_v, sem).wait()
    pltpu.sync_copy(rows_v, out_hbm.at[pl.ds(base, BPW)])


def _route_gather(tab, drug_indices):
    mesh = plsc.VectorSubcoreMesh(core_axis_name="c", subcore_axis_name="s")
    return pl.kernel(
        _sc_gather_body,
        mesh=mesh,
        out_type=jax.ShapeDtypeStruct((B, TABW), jnp.float32),
        scratch_types=[
            pltpu.VMEM((BPW,), jnp.int32),
            pltpu.VMEM((BPW, TABW), jnp.float32),
            pltpu.SemaphoreType.DMA,
        ],
    )(tab, drug_indices)


def _fused_body(x_ref, w1_ref, b1_ref, w2_ref, b2_ref, wp_ref, bp_ref,
                g_ref, out_ref, w1b_ref, w2b_ref, wpf_ref):
    i = pl.program_id(0)

    @pl.when(i == 0)
    def _build_weights():
        w1b_ref[...] = w1_ref[...].astype(jnp.bfloat16)
        w2b_ref[...] = w2_ref[...].astype(jnp.bfloat16)
        for p in range(P):
            wpf_ref[:, p * K:(p + 1) * K] = (
                wp_ref[p].astype(jnp.bfloat16))

    x = x_ref[...].astype(jnp.bfloat16)
    h = jnp.maximum(jnp.dot(x, w1b_ref[...],
                            preferred_element_type=jnp.float32)
                    + b1_ref[0, :], 0.0)
    h = jnp.maximum(jnp.dot(h.astype(jnp.bfloat16), w2b_ref[...],
                            preferred_element_type=jnp.float32)
                    + b2_ref[0, :], 0.0)
    a = jnp.maximum(jnp.dot(h.astype(jnp.bfloat16), wpf_ref[...],
                            preferred_element_type=jnp.float32)
                    + bp_ref[...].reshape(1, P * K), 0.0)

    g = g_ref[...]
    wdg = g[:, :K]
    bdg = g[:, K]
    pwf = g[:, K + 1]

    sel = a[:, :K]
    for p in range(1, P):
        sel = jnp.where((pwf == float(p))[:, None],
                        a[:, p * K:(p + 1) * K], sel)
    out_ref[0, 0, :] = jnp.sum(sel * wdg, axis=1) + bdg


def kernel(x, drug_indices, drug_to_pw, W1, b1, W2, b2, Wp, bp, Wd, bd):
    tab = jnp.concatenate(
        [Wd, bd[:, None], drug_to_pw.astype(jnp.float32)[:, None],
         jnp.zeros((D, TABW - K - 2), jnp.float32)], axis=1)
    g = _route_gather(tab, drug_indices)

    out = pl.pallas_call(
        _fused_body,
        grid=(GRID,),
        in_specs=[
            pl.BlockSpec((BLK, IN), lambda i: (i, 0)),
            pl.BlockSpec((IN, H1), lambda i: (0, 0)),
            pl.BlockSpec((1, H1), lambda i: (0, 0)),
            pl.BlockSpec((H1, H2), lambda i: (0, 0)),
            pl.BlockSpec((1, H2), lambda i: (0, 0)),
            pl.BlockSpec((P, H2, K), lambda i: (0, 0, 0)),
            pl.BlockSpec((P, K), lambda i: (0, 0)),
            pl.BlockSpec((BLK, TABW), lambda i: (i, 0)),
        ],
        out_specs=pl.BlockSpec((1, 1, BLK), lambda i: (i, 0, 0)),
        out_shape=jax.ShapeDtypeStruct((GRID, 1, BLK), jnp.float32),
        scratch_shapes=[
            pltpu.VMEM((IN, H1), jnp.bfloat16),
            pltpu.VMEM((H1, H2), jnp.bfloat16),
            pltpu.VMEM((H2, P * K), jnp.bfloat16),
        ],
    )(x, W1, b1.reshape(1, H1), W2, b2.reshape(1, H2), Wp, bp, g)
    return out.reshape(B)


# speedup vs baseline: 1.0905x; 1.0266x over previous
import functools

import jax
import jax.numpy as jnp
from jax import lax
from jax.experimental import pallas as pl
from jax.experimental.pallas import tpu as pltpu
from jax.experimental.pallas import tpu_sc as plsc

B = 4096
IN = 2048
H1 = 512
H2 = 256
P = 16
K = 128
D = 64

BLK = 512
GRID = B // BLK

TABW = 256
NC = 2
NS = 16
NW = NC * NS
BPW = B // NW


def _sc_gather_body(tab_hbm, idx_hbm, out_hbm, idx_v, rows_v, sem):
    wid = lax.axis_index("s") * NC + lax.axis_index("c")
    base = wid * BPW
    pltpu.sync_copy(idx_hbm.at[pl.ds(base, BPW)], idx_v)
    pltpu.async_copy(tab_hbm.at[idx_v], rows_v, sem).wait()
    pltpu.sync_copy(rows_v, out_hbm.at[pl.ds(base, BPW)])


def _route_gather(tab, drug_indices):
    mesh = plsc.VectorSubcoreMesh(core_axis_name="c", subcore_axis_name="s")
    return pl.kernel(
        _sc_gather_body,
        mesh=mesh,
        out_type=jax.ShapeDtypeStruct((B, TABW), jnp.float32),
        scratch_types=[
            pltpu.VMEM((BPW,), jnp.int32),
            pltpu.VMEM((BPW, TABW), jnp.float32),
            pltpu.SemaphoreType.DMA,
        ],
    )(tab, drug_indices)


def _fused_body(x_ref, w1_ref, b1_ref, w2_ref, b2_ref, wpf_ref,
                bpf_ref, g_ref, out_ref):
    x = x_ref[...].astype(jnp.bfloat16)
    h = jnp.maximum(jnp.dot(x, w1_ref[...].astype(jnp.bfloat16),
                            preferred_element_type=jnp.float32)
                    + b1_ref[0, :], 0.0)
    h = jnp.maximum(jnp.dot(h.astype(jnp.bfloat16),
                            w2_ref[...].astype(jnp.bfloat16),
                            preferred_element_type=jnp.float32)
                    + b2_ref[0, :], 0.0)
    a = jnp.maximum(jnp.dot(h.astype(jnp.bfloat16),
                            wpf_ref[...].astype(jnp.bfloat16),
                            preferred_element_type=jnp.float32)
                    + bpf_ref[0, :], 0.0)

    g = g_ref[...]
    wdg = g[:, :K]
    bdg = g[:, K]
    pwf = g[:, K + 1]

    sel = a[:, :K]
    for p in range(1, P):
        sel = jnp.where((pwf == float(p))[:, None],
                        a[:, p * K:(p + 1) * K], sel)
    out_ref[0, 0, :] = jnp.sum(sel * wdg, axis=1) + bdg


def kernel(x, drug_indices, drug_to_pw, W1, b1, W2, b2, Wp, bp, Wd, bd):
    wpf = Wp.transpose(1, 0, 2).reshape(H2, P * K)
    bpf = bp.reshape(1, P * K)
    tab = jnp.concatenate(
        [Wd, bd[:, None], drug_to_pw.astype(jnp.float32)[:, None],
         jnp.zeros((D, TABW - K - 2), jnp.float32)], axis=1)

    g = _route_gather(tab, drug_indices)

    out = pl.pallas_call(
        _fused_body,
        grid=(GRID,),
        in_specs=[
            pl.BlockSpec((BLK, IN), lambda i: (i, 0)),
            pl.BlockSpec((IN, H1), lambda i: (0, 0)),
            pl.BlockSpec((1, H1), lambda i: (0, 0)),
            pl.BlockSpec((H1, H2), lambda i: (0, 0)),
            pl.BlockSpec((1, H2), lambda i: (0, 0)),
            pl.BlockSpec((H2, P * K), lambda i: (0, 0)),
            pl.BlockSpec((1, P * K), lambda i: (0, 0)),
            pl.BlockSpec((BLK, TABW), lambda i: (i, 0)),
        ],
        out_specs=pl.BlockSpec((1, 1, BLK), lambda i: (i, 0, 0)),
        out_shape=jax.ShapeDtypeStruct((GRID, 1, BLK), jnp.float32),
    )(x, W1, b1.reshape(1, H1), W2, b2.reshape(1, H2), wpf, bpf, g)
    return out.reshape(B)


# speedup vs baseline: 1.0965x; 1.0055x over previous
import jax
import jax.numpy as jnp
from jax import lax
from jax.experimental import pallas as pl
from jax.experimental.pallas import tpu as pltpu
from jax.experimental.pallas import tpu_sc as plsc

B = 4096
IN = 2048
H1 = 512
H2 = 256
P = 16
K = 128
D = 64

BLK = 512
GRID = B // BLK

TABW = 256
NC = 2
NS = 16
NW = NC * NS
BPW = B // NW


def _sc_gather_body(tab_hbm, idx_hbm, out_hbm, idx_v, rows_v, sem):
    wid = lax.axis_index("s") * NC + lax.axis_index("c")
    base = wid * BPW
    pltpu.sync_copy(idx_hbm.at[pl.ds(base, BPW)], idx_v)
    pltpu.async_copy(tab_hbm.at[idx_v], rows_v, sem).wait()
    pltpu.sync_copy(rows_v, out_hbm.at[pl.ds(base, BPW)])


def _route_gather(tab, drug_indices):
    mesh = plsc.VectorSubcoreMesh(core_axis_name="c", subcore_axis_name="s")
    return pl.kernel(
        _sc_gather_body,
        mesh=mesh,
        out_type=jax.ShapeDtypeStruct((B, TABW), jnp.float32),
        scratch_types=[
            pltpu.VMEM((BPW,), jnp.int32),
            pltpu.VMEM((BPW, TABW), jnp.float32),
            pltpu.SemaphoreType.DMA,
        ],
    )(tab, drug_indices)


def _fused_body(x_ref, w1_ref, b1_ref, w2_ref, b2_ref, wpf_ref,
                bpf_ref, g_ref, out_ref):
    x = x_ref[...].astype(jnp.bfloat16)
    h = jnp.maximum(jnp.dot(x, w1_ref[...].astype(jnp.bfloat16),
                            preferred_element_type=jnp.float32)
                    + b1_ref[0, :], 0.0)
    h = jnp.maximum(jnp.dot(h.astype(jnp.bfloat16),
                            w2_ref[...].astype(jnp.bfloat16),
                            preferred_element_type=jnp.float32)
                    + b2_ref[0, :], 0.0)
    a = jnp.maximum(jnp.dot(h.astype(jnp.bfloat16),
                            wpf_ref[...].astype(jnp.bfloat16),
                            preferred_element_type=jnp.float32)
                    + bpf_ref[0, :], 0.0)

    g = g_ref[...]
    wdg = g[:, :K]
    bdg = g[:, K]
    pwf = g[:, K + 1]

    sel = a[:, :K]
    for p in range(1, P):
        sel = jnp.where((pwf == float(p))[:, None],
                        a[:, p * K:(p + 1) * K], sel)
    out_ref[0, 0, :] = jnp.sum(sel * wdg, axis=1) + bdg


def kernel(x, drug_indices, drug_to_pw, W1, b1, W2, b2, Wp, bp, Wd, bd):
    wpf = Wp.transpose(1, 0, 2).reshape(H2, P * K)
    bpf = bp.reshape(1, P * K)
    tab = jnp.concatenate(
        [Wd, bd[:, None], drug_to_pw.astype(jnp.float32)[:, None],
         jnp.zeros((D, TABW - K - 2), jnp.float32)], axis=1)

    g = _route_gather(tab, drug_indices)

    out = pl.pallas_call(
        _fused_body,
        grid=(GRID,),
        in_specs=[
            pl.BlockSpec((BLK, IN), lambda i: (i, 0)),
            pl.BlockSpec((IN, H1), lambda i: (0, 0)),
            pl.BlockSpec((1, H1), lambda i: (0, 0)),
            pl.BlockSpec((H1, H2), lambda i: (0, 0)),
            pl.BlockSpec((1, H2), lambda i: (0, 0)),
            pl.BlockSpec((H2, P * K), lambda i: (0, 0)),
            pl.BlockSpec((1, P * K), lambda i: (0, 0)),
            pl.BlockSpec((BLK, TABW), lambda i: (i, 0)),
        ],
        out_specs=pl.BlockSpec((1, 1, BLK), lambda i: (i, 0, 0)),
        out_shape=jax.ShapeDtypeStruct((GRID, 1, BLK), jnp.float32),
    )(x, W1, b1.reshape(1, H1), W2, b2.reshape(1, H2), wpf, bpf, g)
    return out.reshape(B)
